# baseline (speedup 1.0000x reference)
import functools

import jax
import jax.numpy as jnp
from jax import lax
from jax.experimental import pallas as pl
from jax.experimental.pallas import tpu as pltpu
from jax.experimental.pallas import tpu_sc as plsc

BATCH = 4096
SEQ = 200
DIM = 64
VOCAB = 1000000
ROW2 = 128
PAIR_ROWS = VOCAB // 2
NUM_WORKERS = 32
B_PER_W = BATCH // NUM_WORKERS
LANES = 16
VECS = DIM // LANES
FULL_BLK = VOCAB // ROW2
KEYBITS = 20

_MESH = plsc.VectorSubcoreMesh(core_axis_name="c", subcore_axis_name="s")


def _transpose_block(stage, tr):
    iota = lax.iota(jnp.int32, LANES)
    rv = [(iota + c * LANES) >> 1 for c in range(ROW2 // LANES)]
    cv = [((iota + c * LANES) & 1) * DIM for c in range(ROW2 // LANES)]

    def d_body(d, carry):
        dvec = jnp.full((LANES,), d, jnp.int32)
        for c in range(ROW2 // LANES):
            vals = stage[d, pl.ds(c * LANES, LANES)]
            plsc.store_scatter(tr, [rv[c], cv[c] + dvec], vals)
        return carry

    lax.fori_loop(0, DIM, d_body, 0)


def _format_body(tbl_t, tail_hbm, scratch, stage0, stage1, tr0, tr1,
                 sem0, sem1):
    wid = lax.axis_index("s") * 2 + lax.axis_index("c")

    stages = (stage0, stage1)
    trs = (tr0, tr1)
    sems = (sem0, sem1)

    def src_col(k):
        cb = wid + k * NUM_WORKERS
        return pl.multiple_of(cb * ROW2, ROW2)

    def load_blk(k, p):
        pltpu.async_copy(tbl_t.at[pl.ds(0, DIM), pl.ds(src_col(k), ROW2)],
                         stages[p], sems[p])

    n_mine = (FULL_BLK - wid + NUM_WORKERS - 1) // NUM_WORKERS

    load_blk(0, 0)

    @pl.when(n_mine > 1)
    def _():
        load_blk(1, 1)

    @pl.when(wid == 0)
    def _():
        pltpu.sync_copy(tail_hbm, tr0.at[pl.ds(0, ROW2 // 4)])
        pltpu.sync_copy(tr0.at[pl.ds(0, ROW2 // 4)],
                        scratch.at[pl.ds(FULL_BLK * (ROW2 // 2), ROW2 // 4)])

    def blk_body(i, carry):
        for p in (0, 1):
            k = 2 * i + p

            @pl.when(k < n_mine)
            def _():
                pltpu.make_async_copy(
                    tbl_t.at[pl.ds(0, DIM), pl.ds(0, ROW2)],
                    stages[p], sems[p]).wait()
                _transpose_block(stages[p], trs[p])
                nxt = k + 2

                @pl.when(nxt < n_mine)
                def _():
                    load_blk(nxt, p)

                pltpu.sync_copy(
                    trs[p],
                    scratch.at[pl.ds(
                        pl.multiple_of(src_col(k) // 2, ROW2 // 2),
                        ROW2 // 2)])
        return carry

    lax.fori_loop(0, (n_mine + 1) // 2, blk_body, 0)


@functools.partial(
    pl.kernel,
    mesh=_MESH,
    out_type=jax.ShapeDtypeStruct((PAIR_ROWS, ROW2), jnp.float32),
    scratch_types=[
        pltpu.VMEM((DIM, ROW2), jnp.float32),
        pltpu.VMEM((DIM, ROW2), jnp.float32),
        pltpu.VMEM((DIM, ROW2), jnp.float32),
        pltpu.VMEM((DIM, ROW2), jnp.float32),
        pltpu.SemaphoreType.DMA,
        pltpu.SemaphoreType.DMA,
    ],
    compiler_params=pltpu.CompilerParams(use_tc_tiling_on_sc=True,
                                         needs_layout_passes=False),
)
def _format_sc(tbl_t, tail_hbm, scratch, stage0, stage1, tr0, tr1,
               sem0, sem1):
    _format_body(tbl_t, tail_hbm, scratch, stage0, stage1, tr0, tr1,
                 sem0, sem1)


def _bow_body(idxh_hbm, nev_hbm, table_hbm, out_hbm, idxh_v, nev_v,
              rows0, rows1, out_v, sem0, sem1):
    wid = lax.axis_index("s") * 2 + lax.axis_index("c")

    pltpu.sync_copy(idxh_hbm.at[pl.ds(wid * B_PER_W * SEQ, B_PER_W * SEQ)],
                    idxh_v)
    pltpu.sync_copy(nev_hbm.at[pl.ds(wid * B_PER_W, B_PER_W)], nev_v)

    bufs = (rows0, rows1)
    sems = (sem0, sem1)

    pltpu.async_copy(table_hbm.at[idxh_v.at[pl.ds(0, SEQ)]], rows0, sem0)
    pltpu.async_copy(table_hbm.at[idxh_v.at[pl.ds(SEQ, SEQ)]], rows1, sem1)

    def acc_loop(buf, lo, hi, col0, accs):
        def r_body(r, a):
            return tuple(x + buf[r, pl.ds(col0 + d * LANES, LANES)]
                         for d, x in enumerate(a))
        return lax.fori_loop(lo, hi, r_body, accs)

    def g_body(g, carry):
        nev_vec = nev_v[pl.ds(g * LANES, LANES)]
        for j in range(LANES):
            bb = g * LANES + j
            p = j & 1
            buf, sem = bufs[p], sems[p]
            pltpu.make_async_copy(table_hbm.at[idxh_v.at[pl.ds(0, SEQ)]],
                                  buf, sem).wait()
            n = nev_vec[j]
            accs = tuple(jnp.zeros((LANES,), jnp.float32)
                         for _ in range(VECS))
            accs = acc_loop(buf, 0, n, 0, accs)
            accs = acc_loop(buf, n, SEQ, DIM, accs)
            for d in range(VECS):
                out_v[bb, pl.ds(d * LANES, LANES)] = accs[d]
            nxt = bb + 2

            @pl.when(nxt < B_PER_W)
            def _():
                pltpu.async_copy(
                    table_hbm.at[idxh_v.at[pl.ds(nxt * SEQ, SEQ)]], buf, sem)

        return carry

    lax.fori_loop(0, B_PER_W // LANES, g_body, 0)

    pltpu.sync_copy(out_v, out_hbm.at[pl.ds(wid * B_PER_W, B_PER_W)])


@functools.partial(
    pl.kernel,
    mesh=_MESH,
    out_type=jax.ShapeDtypeStruct((BATCH, DIM), jnp.float32),
    scratch_types=[
        pltpu.VMEM((B_PER_W * SEQ,), jnp.int32),
        pltpu.VMEM((B_PER_W,), jnp.int32),
        pltpu.VMEM((SEQ, ROW2), jnp.float32),
        pltpu.VMEM((SEQ, ROW2), jnp.float32),
        pltpu.VMEM((B_PER_W, DIM), jnp.float32),
        pltpu.SemaphoreType.DMA,
        pltpu.SemaphoreType.DMA,
    ],
    compiler_params=pltpu.CompilerParams(use_tc_tiling_on_sc=True,
                                         needs_layout_passes=False),
)
def _bow_sc(idxh_hbm, nev_hbm, table_hbm, out_hbm, idxh_v, nev_v,
            rows0, rows1, out_v, sem0, sem1):
    _bow_body(idxh_hbm, nev_hbm, table_hbm, out_hbm, idxh_v, nev_v,
              rows0, rows1, out_v, sem0, sem1)


@jax.jit
def kernel(indices, table):
    idx = indices.astype(jnp.int32)
    key = ((idx & 1) << KEYBITS) | idx
    skey = jnp.sort(key, axis=1)
    n_even = (SEQ - jnp.sum(skey >> KEYBITS, axis=1)).astype(jnp.int32)
    idx_half = ((skey & ((1 << KEYBITS) - 1)) >> 1).reshape(-1)

    tail_packed = table[FULL_BLK * ROW2:].reshape(ROW2 // 4, ROW2)
    packed = _format_sc(table.T, tail_packed)
    return _bow_sc(idx_half, n_even, packed)
